# baseline (speedup 1.0000x reference)
import functools

import jax
import jax.numpy as jnp
from jax import lax
from jax.experimental import pallas as pl
from jax.experimental.pallas import tpu as pltpu
from jax.experimental.pallas import tpu_sc as plsc

SEQ = 200
BATCH = 4096
DIM = 64
OUT_DIM = 128
NC = 2
NS = 16
NW = NC * NS
BPW = BATCH // NW
NTR = SEQ // 8
C0 = 128
C1 = SEQ - C0


def _sc_transpose_body(text_hbm, out_hbm, raw_v, idxT_v):
    wid = lax.axis_index("s") * NC + lax.axis_index("c")
    base = wid * BPW
    for tr in range(NTR):
        pltpu.sync_copy(text_hbm.at[pl.ds(tr * 8, 8), pl.ds(base, BPW)],
                        raw_v.at[tr])
    lanes = lax.iota(jnp.int32, 16)

    def trans_tile(tr, carry):
        s0 = tr * 8
        for r in range(8):
            for c in range(BPW // 16):
                dest = (lanes + (c * 16)) * SEQ + (s0 + r)
                plsc.store_scatter(idxT_v, [dest],
                                   raw_v[tr, r, c * 16:(c + 1) * 16])
        return carry

    lax.fori_loop(0, NTR, trans_tile, 0)
    pltpu.sync_copy(idxT_v, out_hbm.at[pl.ds(base * SEQ, BPW * SEQ)])


@jax.jit
def _sc_transpose(text):
    mesh = plsc.VectorSubcoreMesh(core_axis_name="c", subcore_axis_name="s")
    return pl.kernel(
        _sc_transpose_body,
        out_type=jax.ShapeDtypeStruct((BATCH * SEQ,), jnp.int32),
        mesh=mesh,
        scratch_types=[
            pltpu.VMEM((NTR, 8, BPW), jnp.int32),
            pltpu.VMEM((BPW * SEQ,), jnp.int32),
        ],
        compiler_params=pltpu.CompilerParams(use_tc_tiling_on_sc=True,
                                             needs_layout_passes=False),
    )(text)


VOCAB = 1000000
DCH = 200
NCHUNK = VOCAB // DCH
DTRIPS = NCHUNK // NW
DREM = NCHUNK - DTRIPS * NW


def _sc_detile_body(table_hbm, out_hbm, bufs, flats, sems):
    wid = lax.axis_index("s") * NC + lax.axis_index("c")

    def start_read(c, buf):
        pltpu.async_copy(table_hbm.at[pl.ds(c * DCH, DCH), :], bufs.at[buf],
                         sems.at[buf])

    def wait_read(c, buf):
        pltpu.make_async_copy(table_hbm.at[pl.ds(c * DCH, DCH), :],
                              bufs.at[buf], sems.at[buf]).wait()

    def flatten(buf):
        def frow(s, carry):
            o = s * DIM
            flats[buf, pl.ds(o, 16)] = bufs[buf, s, 0:16]
            flats[buf, pl.ds(o + 16, 16)] = bufs[buf, s, 16:32]
            flats[buf, pl.ds(o + 32, 16)] = bufs[buf, s, 32:48]
            flats[buf, pl.ds(o + 48, 16)] = bufs[buf, s, 48:64]
            return carry

        lax.fori_loop(0, DCH, frow, 0, unroll=4)

    def start_write(c, buf):
        pltpu.async_copy(flats.at[buf],
                         out_hbm.at[pl.ds(c * DCH * DIM, DCH * DIM)],
                         sems.at[2 + buf])

    def wait_write(c, buf):
        pltpu.make_async_copy(flats.at[buf],
                              out_hbm.at[pl.ds(c * DCH * DIM, DCH * DIM)],
                              sems.at[2 + buf]).wait()

    def one(c, buf):
        wait_read(c, buf)
        flatten(buf)
        start_write(c, buf)

    start_read(wid, 0)

    c0 = wid
    c1 = wid + NW
    start_read(c1, 1)
    one(c0, 0)

    @pl.when(c1 + NW < NW * DTRIPS)
    def _():
        start_read(c1 + NW, 0)

    one(c1, 1)

    def body2(i, carry):
        c0 = wid + (2 * i) * NW
        c1 = c0 + NW
        start_read(c1, 1)
        wait_write(c0 - 2 * NW, 0)
        one(c0, 0)

        @pl.when(c1 + NW < NW * DTRIPS)
        def _():
            start_read(c1 + NW, 0)

        wait_write(c1 - 2 * NW, 1)
        one(c1, 1)
        return carry

    lax.fori_loop(1, DTRIPS // 2, body2, 0)
    wait_write(wid + (DTRIPS - 2) * NW, 0)
    wait_write(wid + (DTRIPS - 1) * NW, 1)

    @pl.when(wid < DREM)
    def _():
        c = NW * DTRIPS + wid
        start_read(c, 0)
        one(c, 0)
        wait_write(c, 0)


@jax.jit
def _sc_detile(table):
    mesh = plsc.VectorSubcoreMesh(core_axis_name="c", subcore_axis_name="s")
    return pl.kernel(
        _sc_detile_body,
        out_type=jax.ShapeDtypeStruct((VOCAB * DIM,), jnp.float32),
        mesh=mesh,
        scratch_types=[
            pltpu.VMEM((2, DCH, DIM), jnp.float32),
            pltpu.VMEM((2, DCH * DIM), jnp.float32),
            pltpu.SemaphoreType.DMA((4,)),
        ],
        compiler_params=pltpu.CompilerParams(use_tc_tiling_on_sc=True,
                                             needs_layout_passes=False),
    )(table)


def _sc_pool_body(idxT_hbm, table_hbm, out_hbm, idx_v, rows_v, sums_v,
                  sems):
    wid = lax.axis_index("s") * NC + lax.axis_index("c")
    base = wid * BPW
    pltpu.sync_copy(idxT_hbm.at[pl.ds(base * SEQ, BPW * SEQ)], idx_v)

    def gather_col(j, buf):
        pltpu.async_copy(
            table_hbm.at[idx_v.at[pl.ds(j * SEQ, C0)]],
            rows_v.at[buf, pl.ds(0, C0)], sems.at[buf])
        pltpu.async_copy(
            table_hbm.at[idx_v.at[pl.ds(j * SEQ + C0, C1)]],
            rows_v.at[buf, pl.ds(C0, C1)], sems.at[buf])

    def wait_col(j, buf):
        pltpu.make_async_copy(
            table_hbm.at[idx_v.at[pl.ds(j * SEQ, C0)]],
            rows_v.at[buf, pl.ds(0, C0)], sems.at[buf]).wait()
        pltpu.make_async_copy(
            table_hbm.at[idx_v.at[pl.ds(j * SEQ + C0, C1)]],
            rows_v.at[buf, pl.ds(C0, C1)], sems.at[buf]).wait()

    def accum_col(j, buf):
        def srow(s, acc):
            a0, a1, a2, a3 = acc
            return (a0 + rows_v[buf, s, 0:16], a1 + rows_v[buf, s, 16:32],
                    a2 + rows_v[buf, s, 32:48], a3 + rows_v[buf, s, 48:64])

        z = jnp.zeros((16,), jnp.float32)
        a0, a1, a2, a3 = lax.fori_loop(0, SEQ, srow, (z, z, z, z),
                                       unroll=8)
        scale = jnp.float32(1.0 / SEQ)
        sums_v[j, 0:16] = a0 * scale
        sums_v[j, 16:32] = a1 * scale
        sums_v[j, 32:48] = a2 * scale
        sums_v[j, 48:64] = a3 * scale

    gather_col(0, 0)

    def pair(i, carry):
        j = 2 * i
        gather_col(j + 1, 1)
        wait_col(j, 0)
        accum_col(j, 0)

        @pl.when(j + 2 < BPW)
        def _():
            gather_col(j + 2, 0)

        wait_col(j + 1, 1)
        accum_col(j + 1, 1)
        return carry

    lax.fori_loop(0, BPW // 2, pair, 0)
    pltpu.sync_copy(sums_v, out_hbm.at[pl.ds(base, BPW)])


@jax.jit
def _sc_pool(idxT, tabflat):
    mesh = plsc.VectorSubcoreMesh(core_axis_name="c", subcore_axis_name="s")
    return pl.kernel(
        _sc_pool_body,
        out_type=jax.ShapeDtypeStruct((BATCH, DIM), jnp.float32),
        mesh=mesh,
        scratch_types=[
            pltpu.VMEM((BPW * SEQ,), jnp.int32),
            pltpu.VMEM((2, SEQ, DIM), jnp.float32),
            pltpu.VMEM((BPW, DIM), jnp.float32),
            pltpu.SemaphoreType.DMA((2,)),
        ],
        compiler_params=pltpu.CompilerParams(use_tc_tiling_on_sc=False,
                                             needs_layout_passes=False),
    )(idxT, tabflat)


def _tc_fc_body(x_ref, w_ref, b_ref, o_ref):
    o_ref[...] = lax.dot_general(
        x_ref[...], w_ref[...], (((1,), (1,)), ((), ())),
        preferred_element_type=jnp.float32) + b_ref[...]


@jax.jit
def _tc_fc(pooled, W, b2d):
    return pl.pallas_call(
        _tc_fc_body,
        out_shape=jax.ShapeDtypeStruct((BATCH, OUT_DIM), jnp.float32),
    )(pooled, W, b2d)


def kernel(text, emb_table, W, b):
    idxT = _sc_transpose(text.astype(jnp.int32))
    tabflat = _sc_detile(emb_table)
    pooled = _sc_pool(idxT, tabflat.reshape(VOCAB, DIM))
    return _tc_fc(pooled, W, b.reshape(1, OUT_DIM))
